# baseline (speedup 1.0000x reference)
import functools

import jax
import jax.numpy as jnp
from jax import lax
from jax.experimental import pallas as pl
from jax.experimental.pallas import tpu as pltpu
from jax.experimental.pallas import tpu_sc as plsc

_F32 = jnp.float32


def _sc_gather(node2h, edge16, nidx2, nb_rows, eg_rows):
    fn = node2h.shape[1]
    fw = edge16.shape[1]
    mesh = plsc.VectorSubcoreMesh(core_axis_name="c", subcore_axis_name="s")
    nw = 32
    n_chunks_n = nb_rows // (nw * 200)
    n_chunks_e = eg_rows // (nw * 200)

    @functools.partial(
        pl.kernel,
        mesh=mesh,
        out_type=[
            jax.ShapeDtypeStruct((nb_rows, fn), jnp.bfloat16),
            jax.ShapeDtypeStruct((eg_rows, fw), jnp.bfloat16),
        ],
        scratch_types=[
            pltpu.VMEM((2, 100), jnp.int32),
            pltpu.VMEM((200, fn), jnp.bfloat16),
            pltpu.VMEM((2, 100), jnp.int32),
            pltpu.VMEM((200, fw), jnp.bfloat16),
            pltpu.SemaphoreType.DMA,
        ],
        compiler_params=pltpu.CompilerParams(use_tc_tiling_on_sc=False),
    )
    def k(node_hbm, edge_hbm, nidx_hbm, nj_out, eg_out,
          idxn_v, nbuf, idxe_v, ebuf, sem):
        wid = lax.axis_index("s") * 2 + lax.axis_index("c")

        def gather_chunks(table, out, idx_v, buf, n_chunks):
            def body(c, carry):
                irow = wid * (2 * n_chunks) + c * 2
                pltpu.sync_copy(nidx_hbm.at[pl.ds(irow, 2)], idx_v)
                cps = [
                    pltpu.async_copy(table.at[idx_v.at[j]],
                                     buf.at[pl.ds(j * 100, 100)], sem)
                    for j in range(2)
                ]
                for cp in cps:
                    cp.wait()
                pltpu.sync_copy(buf, out.at[pl.ds(irow * 100, 200)])
                return carry
            lax.fori_loop(0, n_chunks, body, 0)

        gather_chunks(node_hbm, nj_out, idxn_v, nbuf, n_chunks_n)
        gather_chunks(edge_hbm, eg_out, idxe_v, ebuf, n_chunks_e)

    return k(node2h, edge16, nidx2)


def _tc_dense(node2, edge2, njall, eg16, mask2, W2T, b2, W3iT, W3jT, W3kT,
              W3eT, W3q4, b3, g2, be2, g3, be3, gs, bes, Bd, At, Nbr):
    Fn = node2.shape[1]
    Fe = edge2.shape[1]
    G = 2 * Fe
    G4 = 4 * G
    R = At * Nbr
    RT = Bd * R
    NQ = Nbr // 4
    CH = R
    n2 = float(RT)
    n3 = float(RT * (Nbr - 1))
    eps = 1e-5

    def dot(a, b):
        return lax.dot_general(a, b, (((1,), (0,)), ((), ())),
                               preferred_element_type=_F32)

    def body(node_r, edge_r, mask_r, w2t_r, b2_r, w3it_r, w3jt_r,
             w3kt_r, w3et_r, w3q4_r, b3_r, g2_r, be2_r, g3_r, be3_r,
             gs_r, bes_r, njall_r, eg_r, out_r, tbr_scr, c2_scr, ebuf,
             njbuf, njtbuf, sems):
        w2t = w2t_r[...]
        w3it = w3it_r[...]
        w3jt = w3jt_r[...]
        w3kt = w3kt_r[...]
        w3et = w3et_r[...]
        w3q4 = w3q4_r[...]
        b2v = b2_r[...]
        b3v = b3_r[...]

        rowj = lax.broadcasted_iota(jnp.int32, (R, 1), 0) % Nbr
        laneq = lax.broadcasted_iota(jnp.int32, (R, G4), 1) // G

        def batch_copies(b):
            sl = b % 2
            return [
                pltpu.make_async_copy(njall_r.at[pl.ds(b * R, R)],
                                      njbuf.at[sl], sems.at[sl]),
                pltpu.make_async_copy(njall_r.at[pl.ds(RT + b * R, R)],
                                      njtbuf.at[sl], sems.at[2 + sl]),
                pltpu.make_async_copy(eg_r.at[pl.ds(b * CH, CH)],
                                      ebuf.at[sl], sems.at[4 + sl]),
            ]

        def start_batch(b):
            for cp in batch_copies(b):
                cp.start()

        def wait_batch(b):
            for cp in batch_copies(b):
                cp.wait()

        def compute_s(b, nj):
            nodes = node_r[pl.ds(b * At, At)]
            pi = dot(nodes, w3it)
            pib = jnp.broadcast_to(pi[:, None, :], (At, Nbr, G)).reshape(R, G)
            pe = dot(edge_r[pl.ds(b * R, R)], w3et)
            return pib + dot(nj, w3jt) + pe + b3v

        def grid_x4(sl, mq, s4, pnkt):
            egq = ebuf[sl, :, pl.ds(mq * 4 * Fe, 4 * Fe)]
            epg4 = dot(egq, w3q4)
            ps = []
            for q in range(4):
                p = pnkt[(4 * mq + q) * At:(4 * mq + q + 1) * At]
                ps.append(jnp.broadcast_to(
                    p[:, None, :], (At, Nbr, G)).reshape(R, G))
            return epg4 + s4 + jnp.concatenate(ps, axis=1)

        def keep4(mq):
            return (laneq + 4 * mq != rowj).astype(_F32)

        z1 = jnp.zeros((1, G), _F32)
        z4 = jnp.zeros((1, G4), _F32)
        S12, S22, S13c, S23c = z1, z1, z4, z4
        start_batch(0)
        for b in range(Bd):
            if b + 1 < Bd:
                start_batch(b + 1)
            wait_batch(b)
            sl = b % 2
            nj = njbuf[sl].astype(_F32)
            nodes = node_r[pl.ds(b * At, At)]
            ni = jnp.broadcast_to(nodes[:, None, :], (At, Nbr, Fn))
            ni = ni.reshape(R, Fn)
            mb = jnp.broadcast_to(
                mask_r[pl.ds(b * At, At)][:, :, None], (At, Nbr, Fn)
            ).reshape(R, Fn)
            c2 = dot(ni * nj * mb, w2t) + b2v
            c2_scr[pl.ds(b * R, R)] = c2
            S12 = S12 + jnp.sum(c2, 0, keepdims=True)
            S22 = S22 + jnp.sum(c2 * c2, 0, keepdims=True)
            s = compute_s(b, njbuf[sl])
            s4 = jnp.concatenate([s, s, s, s], axis=1)
            pnkt = dot(njtbuf[sl], w3kt)
            for mq in range(NQ):
                x = grid_x4(sl, mq, s4, pnkt)
                xk = x * keep4(mq)
                S13c = S13c + jnp.sum(xk, 0, keepdims=True)
                S23c = S23c + jnp.sum(x * xk, 0, keepdims=True)
        S13 = (S13c[:, :G] + S13c[:, G:2 * G] + S13c[:, 2 * G:3 * G]
               + S13c[:, 3 * G:])
        S23 = (S23c[:, :G] + S23c[:, G:2 * G] + S23c[:, 2 * G:3 * G]
               + S23c[:, 3 * G:])

        m2 = S12 / n2
        v2 = S22 / n2 - m2 * m2
        sc2 = g2_r[...] * lax.rsqrt(v2 + eps)
        sh2 = be2_r[...] - m2 * sc2
        m3 = S13 / n3
        v3 = S23 / n3 - m3 * m3
        sc3 = g3_r[...] * lax.rsqrt(v3 + eps)
        sh3 = be3_r[...] - m3 * sc3
        sc3t = jnp.concatenate([sc3] * 4, axis=1)
        sh3t = jnp.concatenate([sh3] * 4, axis=1)

        zs = jnp.zeros((1, Fe), _F32)
        S1s, S2s = zs, zs
        start_batch(0)
        for b in range(Bd):
            if b + 1 < Bd:
                start_batch(b + 1)
            wait_batch(b)
            sl = b % 2
            s = compute_s(b, njbuf[sl])
            s4 = jnp.concatenate([s, s, s, s], axis=1)
            pnkt = dot(njtbuf[sl], w3kt)
            tb = jnp.zeros((R, Fe), _F32)
            for mq in range(NQ):
                xn = grid_x4(sl, mq, s4, pnkt) * sc3t + sh3t
                for q in range(4):
                    gate = xn[:, q * G:q * G + Fe]
                    ext = xn[:, q * G + Fe:(q + 1) * G]
                    kq = (4 * mq + q != rowj).astype(_F32)
                    tb = tb + jax.nn.sigmoid(gate) * jnp.tanh(ext) * kq
            tbr_scr[pl.ds(b * R, R)] = tb
            S1s = S1s + jnp.sum(tb, 0, keepdims=True)
            S2s = S2s + jnp.sum(tb * tb, 0, keepdims=True)

        ms = S1s / n2
        vs = S2s / n2 - ms * ms
        scs = gs_r[...] * lax.rsqrt(vs + eps)
        shs = bes_r[...] - ms * scs

        def phase_c(b, carry):
            rows = pl.ds(b * R, R)
            c2n = c2_scr[rows] * sc2 + sh2
            twob = jax.nn.sigmoid(c2n[:, :Fe]) * jnp.tanh(c2n[:, Fe:])
            tb = tbr_scr[rows] * scs + shs
            out_r[rows] = jnp.tanh(edge_r[rows] + twob + tb)
            return carry

        lax.fori_loop(0, Bd, phase_c, 0)

    aspec = pl.BlockSpec(memory_space=pl.ANY)
    return pl.pallas_call(
        body,
        out_shape=jax.ShapeDtypeStruct((RT, Fe), _F32),
        in_specs=[pl.BlockSpec()] * 17 + [aspec, aspec],
        out_specs=pl.BlockSpec(),
        scratch_shapes=[
            pltpu.VMEM((RT, Fe), _F32),
            pltpu.VMEM((RT, G), _F32),
            pltpu.VMEM((2, R, Nbr * Fe), jnp.bfloat16),
            pltpu.VMEM((2, R, Fn), jnp.bfloat16),
            pltpu.VMEM((2, R, Fn), jnp.bfloat16),
            pltpu.SemaphoreType.DMA((6,)),
        ],
        compiler_params=pltpu.CompilerParams(
            vmem_limit_bytes=100 * 1024 * 1024),
    )(node2, edge2, mask2, W2T, b2, W3iT, W3jT, W3kT, W3eT, W3q4,
      b3, g2, be2, g3, be3, gs, bes, njall, eg16)


def kernel(node_embedding, edge_embedding, nbr_idx, nbr_mask, W2, b2, W3,
           b3, g2, be2, g3, be3, gs, bes):
    Bd, At, Nbr, Fe = edge_embedding.shape
    Fn = node_embedding.shape[-1]
    G = 2 * Fe
    R = At * Nbr
    RT = Bd * R
    NQ = Nbr // 4

    node2 = node_embedding.reshape(Bd * At, Fn)
    edge2 = edge_embedding.reshape(RT, Fe)
    node2h = node2.astype(jnp.bfloat16)
    edge16 = edge_embedding.reshape(Bd * At, Nbr * Fe).astype(jnp.bfloat16)
    idx = nbr_idx.reshape(Bd, At, Nbr).astype(jnp.int32)
    off = (jnp.arange(Bd, dtype=jnp.int32) * At)[:, None, None]
    gidx = (idx + off).reshape(Bd, R)
    gidx_t = (jnp.transpose(idx, (0, 2, 1)) + off).reshape(Bd, R)
    nidx = jnp.concatenate([gidx.reshape(RT), gidx_t.reshape(RT)])
    nidx2 = nidx.reshape(2 * RT // 100, 100)

    njall, eg16 = _sc_gather(node2h, edge16, nidx2, 2 * RT, RT)

    mask2 = nbr_mask.reshape(Bd * At, Nbr).astype(_F32)
    W3iT = W3[:, :Fn].T
    W3jT = W3[:, Fn:2 * Fn].T
    W3kT = W3[:, 2 * Fn:3 * Fn].T
    W3eT = W3[:, 3 * Fn:3 * Fn + Fe].T
    W3qT = W3[:, 3 * Fn + Fe:].T
    zb = jnp.zeros((Fe, G), _F32)
    W3q4 = jnp.block([[W3qT if i == j else zb for j in range(4)]
                      for i in range(4)]).astype(jnp.bfloat16)
    W3jT = W3jT.astype(jnp.bfloat16)
    W3kT = W3kT.astype(jnp.bfloat16)

    out2 = _tc_dense(node2, edge2, njall, eg16, mask2, W2.T,
                     b2.reshape(1, G), W3iT, W3jT, W3kT, W3eT, W3q4,
                     b3.reshape(1, G), g2.reshape(1, G), be2.reshape(1, G),
                     g3.reshape(1, G), be3.reshape(1, G),
                     gs.reshape(1, Fe), bes.reshape(1, Fe), Bd, At, Nbr)
    return out2.reshape(Bd, At, Nbr, Fe)


# speedup vs baseline: 1.1191x; 1.1191x over previous
import functools

import jax
import jax.numpy as jnp
from jax import lax
from jax.experimental import pallas as pl
from jax.experimental.pallas import tpu as pltpu
from jax.experimental.pallas import tpu_sc as plsc

_F32 = jnp.float32


def _sc_gather(node2h, edge16, nidx2, nb_rows, eg_rows):
    fn = node2h.shape[1]
    fw = edge16.shape[1]
    mesh = plsc.VectorSubcoreMesh(core_axis_name="c", subcore_axis_name="s")
    nw = 32
    n_chunks_n = nb_rows // (nw * 200)
    n_chunks_e = eg_rows // (nw * 200)

    @functools.partial(
        pl.kernel,
        mesh=mesh,
        out_type=[
            jax.ShapeDtypeStruct((nb_rows, fn), _F32),
            jax.ShapeDtypeStruct((eg_rows, fw), _F32),
        ],
        scratch_types=[
            pltpu.VMEM((2, 100), jnp.int32),
            pltpu.VMEM((200, fn), _F32),
            pltpu.VMEM((2, 100), jnp.int32),
            pltpu.VMEM((200, fw), _F32),
            pltpu.SemaphoreType.DMA,
        ],
        compiler_params=pltpu.CompilerParams(use_tc_tiling_on_sc=False),
    )
    def k(node_hbm, edge_hbm, nidx_hbm, nj_out, eg_out,
          idxn_v, nbuf, idxe_v, ebuf, sem):
        wid = lax.axis_index("s") * 2 + lax.axis_index("c")

        def gather_chunks(table, out, idx_v, buf, n_chunks):
            def body(c, carry):
                irow = wid * (2 * n_chunks) + c * 2
                pltpu.sync_copy(nidx_hbm.at[pl.ds(irow, 2)], idx_v)
                cps = [
                    pltpu.async_copy(table.at[idx_v.at[j]],
                                     buf.at[pl.ds(j * 100, 100)], sem)
                    for j in range(2)
                ]
                for cp in cps:
                    cp.wait()
                pltpu.sync_copy(buf, out.at[pl.ds(irow * 100, 200)])
                return carry
            lax.fori_loop(0, n_chunks, body, 0)

        gather_chunks(node_hbm, nj_out, idxn_v, nbuf, n_chunks_n)
        gather_chunks(edge_hbm, eg_out, idxe_v, ebuf, n_chunks_e)

    return k(node2h, edge16, nidx2)


def _tc_dense(node2, edge2, njall, eg16, mask2, W2T, b2, W3iT, W3jT, W3kT,
              W3eT, W3q4, b3, g2, be2, g3, be3, gs, bes, Bd, At, Nbr):
    Fn = node2.shape[1]
    Fe = edge2.shape[1]
    G = 2 * Fe
    G4 = 4 * G
    R = At * Nbr
    RT = Bd * R
    NQ = Nbr // 4
    CH = R
    n2 = float(RT)
    n3 = float(RT * (Nbr - 1))
    eps = 1e-5

    def dot(a, b):
        return lax.dot_general(a, b, (((1,), (0,)), ((), ())),
                               preferred_element_type=_F32)

    def body(node_r, edge_r, mask_r, w2t_r, b2_r, w3it_r, w3jt_r,
             w3kt_r, w3et_r, w3q4_r, b3_r, g2_r, be2_r, g3_r, be3_r,
             gs_r, bes_r, njall_r, eg_r, out_r, tbr_scr, c2_scr, ebuf,
             njbuf, njtbuf, sems):
        w2t = w2t_r[...]
        w3it = w3it_r[...]
        w3jt = w3jt_r[...]
        w3kt = w3kt_r[...]
        w3et = w3et_r[...]
        w3q4 = w3q4_r[...]
        b2v = b2_r[...]
        b3v = b3_r[...]

        rowj = lax.broadcasted_iota(jnp.int32, (R, 1), 0) % Nbr
        laneq = lax.broadcasted_iota(jnp.int32, (R, G4), 1) // G

        def batch_copies(b):
            sl = b % 2
            return [
                pltpu.make_async_copy(njall_r.at[pl.ds(b * R, R)],
                                      njbuf.at[sl], sems.at[sl]),
                pltpu.make_async_copy(njall_r.at[pl.ds(RT + b * R, R)],
                                      njtbuf.at[sl], sems.at[2 + sl]),
                pltpu.make_async_copy(eg_r.at[pl.ds(b * CH, CH)],
                                      ebuf.at[sl], sems.at[4 + sl]),
            ]

        def start_batch(b):
            for cp in batch_copies(b):
                cp.start()

        def wait_batch(b):
            for cp in batch_copies(b):
                cp.wait()

        def compute_s(b, nj):
            nodes = node_r[pl.ds(b * At, At)]
            pi = dot(nodes, w3it)
            pib = jnp.broadcast_to(pi[:, None, :], (At, Nbr, G)).reshape(R, G)
            pe = dot(edge_r[pl.ds(b * R, R)], w3et)
            return pib + dot(nj, w3jt) + pe + b3v

        def grid_x4(sl, mq, s4, pnkt):
            egq = ebuf[sl, :, pl.ds(mq * 4 * Fe, 4 * Fe)]
            epg4 = dot(egq, w3q4)
            ps = []
            for q in range(4):
                p = pnkt[(4 * mq + q) * At:(4 * mq + q + 1) * At]
                ps.append(jnp.broadcast_to(
                    p[:, None, :], (At, Nbr, G)).reshape(R, G))
            return epg4 + s4 + jnp.concatenate(ps, axis=1)

        def keep4(mq):
            return (laneq + 4 * mq != rowj).astype(_F32)

        z1 = jnp.zeros((1, G), _F32)
        z4 = jnp.zeros((1, G4), _F32)
        S12, S22, S13c, S23c = z1, z1, z4, z4
        start_batch(0)
        for b in range(Bd):
            if b + 1 < Bd:
                start_batch(b + 1)
            wait_batch(b)
            sl = b % 2
            nj = njbuf[sl]
            nodes = node_r[pl.ds(b * At, At)]
            ni = jnp.broadcast_to(nodes[:, None, :], (At, Nbr, Fn))
            ni = ni.reshape(R, Fn)
            mb = jnp.broadcast_to(
                mask_r[pl.ds(b * At, At)][:, :, None], (At, Nbr, Fn)
            ).reshape(R, Fn)
            c2 = dot(ni * nj * mb, w2t) + b2v
            c2_scr[pl.ds(b * R, R)] = c2
            S12 = S12 + jnp.sum(c2, 0, keepdims=True)
            S22 = S22 + jnp.sum(c2 * c2, 0, keepdims=True)
            s = compute_s(b, njbuf[sl])
            s4 = jnp.concatenate([s, s, s, s], axis=1)
            pnkt = dot(njtbuf[sl], w3kt)
            for mq in range(NQ):
                x = grid_x4(sl, mq, s4, pnkt)
                xk = x * keep4(mq)
                S13c = S13c + jnp.sum(xk, 0, keepdims=True)
                S23c = S23c + jnp.sum(x * xk, 0, keepdims=True)
        S13 = (S13c[:, :G] + S13c[:, G:2 * G] + S13c[:, 2 * G:3 * G]
               + S13c[:, 3 * G:])
        S23 = (S23c[:, :G] + S23c[:, G:2 * G] + S23c[:, 2 * G:3 * G]
               + S23c[:, 3 * G:])

        m2 = S12 / n2
        v2 = S22 / n2 - m2 * m2
        sc2 = g2_r[...] * lax.rsqrt(v2 + eps)
        sh2 = be2_r[...] - m2 * sc2
        m3 = S13 / n3
        v3 = S23 / n3 - m3 * m3
        sc3 = g3_r[...] * lax.rsqrt(v3 + eps)
        sh3 = be3_r[...] - m3 * sc3
        sc3t = jnp.concatenate([sc3] * 4, axis=1)
        sh3t = jnp.concatenate([sh3] * 4, axis=1)

        zs = jnp.zeros((1, Fe), _F32)
        S1s, S2s = zs, zs
        start_batch(0)
        for b in range(Bd):
            if b + 1 < Bd:
                start_batch(b + 1)
            wait_batch(b)
            sl = b % 2
            s = compute_s(b, njbuf[sl])
            s4 = jnp.concatenate([s, s, s, s], axis=1)
            pnkt = dot(njtbuf[sl], w3kt)
            tb = jnp.zeros((R, Fe), _F32)
            for mq in range(NQ):
                xn = grid_x4(sl, mq, s4, pnkt) * sc3t + sh3t
                for q in range(4):
                    gate = xn[:, q * G:q * G + Fe]
                    ext = xn[:, q * G + Fe:(q + 1) * G]
                    kq = (4 * mq + q != rowj).astype(_F32)
                    tb = tb + jax.nn.sigmoid(gate) * jnp.tanh(ext) * kq
            tbr_scr[pl.ds(b * R, R)] = tb
            S1s = S1s + jnp.sum(tb, 0, keepdims=True)
            S2s = S2s + jnp.sum(tb * tb, 0, keepdims=True)

        ms = S1s / n2
        vs = S2s / n2 - ms * ms
        scs = gs_r[...] * lax.rsqrt(vs + eps)
        shs = bes_r[...] - ms * scs

        def phase_c(b, carry):
            rows = pl.ds(b * R, R)
            c2n = c2_scr[rows] * sc2 + sh2
            twob = jax.nn.sigmoid(c2n[:, :Fe]) * jnp.tanh(c2n[:, Fe:])
            tb = tbr_scr[rows] * scs + shs
            out_r[rows] = jnp.tanh(edge_r[rows] + twob + tb)
            return carry

        lax.fori_loop(0, Bd, phase_c, 0)

    aspec = pl.BlockSpec(memory_space=pl.ANY)
    return pl.pallas_call(
        body,
        out_shape=jax.ShapeDtypeStruct((RT, Fe), _F32),
        in_specs=[pl.BlockSpec()] * 17 + [aspec, aspec],
        out_specs=pl.BlockSpec(),
        scratch_shapes=[
            pltpu.VMEM((RT, Fe), _F32),
            pltpu.VMEM((RT, G), _F32),
            pltpu.VMEM((2, R, Nbr * Fe), _F32),
            pltpu.VMEM((2, R, Fn), _F32),
            pltpu.VMEM((2, R, Fn), _F32),
            pltpu.SemaphoreType.DMA((6,)),
        ],
        compiler_params=pltpu.CompilerParams(
            vmem_limit_bytes=100 * 1024 * 1024),
    )(node2, edge2, mask2, W2T, b2, W3iT, W3jT, W3kT, W3eT, W3q4,
      b3, g2, be2, g3, be3, gs, bes, njall, eg16)


def kernel(node_embedding, edge_embedding, nbr_idx, nbr_mask, W2, b2, W3,
           b3, g2, be2, g3, be3, gs, bes):
    Bd, At, Nbr, Fe = edge_embedding.shape
    Fn = node_embedding.shape[-1]
    G = 2 * Fe
    R = At * Nbr
    RT = Bd * R
    NQ = Nbr // 4

    node2 = node_embedding.reshape(Bd * At, Fn)
    edge2 = edge_embedding.reshape(RT, Fe)
    edge16 = edge_embedding.reshape(Bd * At, Nbr * Fe)
    idx = nbr_idx.reshape(Bd, At, Nbr).astype(jnp.int32)
    off = (jnp.arange(Bd, dtype=jnp.int32) * At)[:, None, None]
    gidx = (idx + off).reshape(Bd, R)
    gidx_t = (jnp.transpose(idx, (0, 2, 1)) + off).reshape(Bd, R)
    nidx = jnp.concatenate([gidx.reshape(RT), gidx_t.reshape(RT)])
    nidx2 = nidx.reshape(2 * RT // 100, 100)

    njall, eg16 = _sc_gather(node2, edge16, nidx2, 2 * RT, RT)

    mask2 = nbr_mask.reshape(Bd * At, Nbr).astype(_F32)
    W3iT = W3[:, :Fn].T
    W3jT = W3[:, Fn:2 * Fn].T
    W3kT = W3[:, 2 * Fn:3 * Fn].T
    W3eT = W3[:, 3 * Fn:3 * Fn + Fe].T
    W3qT = W3[:, 3 * Fn + Fe:].T
    zb = jnp.zeros((Fe, G), _F32)
    W3q4 = jnp.block([[W3qT if i == j else zb for j in range(4)]
                      for i in range(4)])

    out2 = _tc_dense(node2, edge2, njall, eg16, mask2, W2.T,
                     b2.reshape(1, G), W3iT, W3jT, W3kT, W3eT, W3q4,
                     b3.reshape(1, G), g2.reshape(1, G), be2.reshape(1, G),
                     g3.reshape(1, G), be3.reshape(1, G),
                     gs.reshape(1, Fe), bes.reshape(1, Fe), Bd, At, Nbr)
    return out2.reshape(Bd, At, Nbr, Fe)


# speedup vs baseline: 1.3178x; 1.1775x over previous
import functools

import jax
import jax.numpy as jnp
from jax import lax
from jax.experimental import pallas as pl
from jax.experimental.pallas import tpu as pltpu
from jax.experimental.pallas import tpu_sc as plsc

_F32 = jnp.float32
_BF16 = jnp.bfloat16


def _sc_gather(node2, edge_perm4, nidx2, eidx2, nb_rows, eg_rows):
    fn = node2.shape[1]
    f4 = edge_perm4.shape[1]
    mesh = plsc.VectorSubcoreMesh(core_axis_name="c", subcore_axis_name="s")
    nw = 32
    n_chunks_n = nb_rows // (nw * 400)
    n_chunks_e = eg_rows // (nw * 400)

    @functools.partial(
        pl.kernel,
        mesh=mesh,
        out_type=[
            jax.ShapeDtypeStruct((nb_rows, fn), _F32),
            jax.ShapeDtypeStruct((eg_rows, f4), _F32),
        ],
        scratch_types=[
            pltpu.VMEM((4, 100), jnp.int32),
            pltpu.VMEM((400, fn), _F32),
            pltpu.VMEM((4, 100), jnp.int32),
            pltpu.VMEM((400, f4), _F32),
            pltpu.SemaphoreType.DMA,
        ],
        compiler_params=pltpu.CompilerParams(use_tc_tiling_on_sc=False),
    )
    def k(node_hbm, edge_hbm, nidx_hbm, eidx_hbm, nj_out, eg_out,
          idxn_v, nbuf, idxe_v, ebuf, sem):
        wid = lax.axis_index("s") * 2 + lax.axis_index("c")

        def gather_chunks(table, idx_hbm, out, idx_v, buf, n_chunks):
            def body(c, carry):
                irow = wid * (4 * n_chunks) + c * 4
                pltpu.sync_copy(idx_hbm.at[pl.ds(irow, 4)], idx_v)
                cps = [
                    pltpu.async_copy(table.at[idx_v.at[j]],
                                     buf.at[pl.ds(j * 100, 100)], sem)
                    for j in range(4)
                ]
                for cp in cps:
                    cp.wait()
                pltpu.sync_copy(buf, out.at[pl.ds(irow * 100, 400)])
                return carry
            lax.fori_loop(0, n_chunks, body, 0)

        gather_chunks(node_hbm, nidx_hbm, nj_out, idxn_v, nbuf, n_chunks_n)
        gather_chunks(edge_hbm, eidx_hbm, eg_out, idxe_v, ebuf, n_chunks_e)

    return k(node2, edge_perm4, nidx2, eidx2)


def _tc_dense(node2, edge2, njall, eg4, mask2, W2T, b2, W3iT, W3jT, W3kT,
              W3eT, W3q4, b3, g2, be2, g3, be3, gs, bes, Bd, At, Nbr):
    Fn = node2.shape[1]
    Fe = edge2.shape[1]
    G = 2 * Fe
    G4 = 4 * G
    R = At * Nbr
    RT = Bd * R
    NQ = Nbr // 4
    CH = NQ * R
    n2 = float(RT)
    n3 = float(RT * (Nbr - 1))
    eps = 1e-5

    def dot(a, b):
        return lax.dot_general(a, b, (((1,), (0,)), ((), ())),
                               preferred_element_type=_F32)

    def body(node_r, edge_r, mask_r, w2t_r, b2_r, w3it_r, w3jt_r,
             w3kt_r, w3et_r, w3q4_r, b3_r, g2_r, be2_r, g3_r, be3_r,
             gs_r, bes_r, njall_r, eg_r, out_r, tbr_scr, c2_scr, ebuf,
             njbuf, njtbuf, sems):
        w2t = w2t_r[...]
        w3it = w3it_r[...]
        w3jt = w3jt_r[...]
        w3kt = w3kt_r[...]
        w3et = w3et_r[...]
        w3q4 = w3q4_r[...]
        b2v = b2_r[...]
        b3v = b3_r[...]

        rowj = lax.broadcasted_iota(jnp.int32, (R, 1), 0) % Nbr
        laneq = lax.broadcasted_iota(jnp.int32, (R, G4), 1) // G

        def batch_copies(b):
            sl = b % 2
            return [
                pltpu.make_async_copy(njall_r.at[pl.ds(b * R, R)],
                                      njbuf.at[sl], sems.at[sl]),
                pltpu.make_async_copy(njall_r.at[pl.ds(RT + b * R, R)],
                                      njtbuf.at[sl], sems.at[2 + sl]),
                pltpu.make_async_copy(eg_r.at[pl.ds(b * CH, CH)],
                                      ebuf.at[sl], sems.at[4 + sl]),
            ]

        def start_batch(b):
            for cp in batch_copies(b):
                cp.start()

        def wait_batch(b):
            for cp in batch_copies(b):
                cp.wait()

        def compute_s(b, nj):
            nodes = node_r[pl.ds(b * At, At)]
            pi = dot(nodes, w3it)
            pib = jnp.broadcast_to(pi[:, None, :], (At, Nbr, G)).reshape(R, G)
            pe = dot(edge_r[pl.ds(b * R, R)], w3et)
            return pib + dot(nj.astype(_BF16), w3jt) + pe + b3v

        def grid_x4(sl, mq, s4, pnkt):
            egq = ebuf[sl, pl.ds(mq * R, R), :]
            epg4 = dot(egq.astype(_BF16), w3q4)
            ps = []
            for q in range(4):
                p = pnkt[(4 * mq + q) * At:(4 * mq + q + 1) * At]
                ps.append(jnp.broadcast_to(
                    p[:, None, :], (At, Nbr, G)).reshape(R, G))
            return epg4 + s4 + jnp.concatenate(ps, axis=1)

        def keep4(mq):
            return (laneq + 4 * mq != rowj).astype(_F32)

        z1 = jnp.zeros((1, G), _F32)
        z4 = jnp.zeros((1, G4), _F32)
        S12, S22, S13c, S23c = z1, z1, z4, z4
        start_batch(0)
        for b in range(Bd):
            if b + 1 < Bd:
                start_batch(b + 1)
            wait_batch(b)
            sl = b % 2
            nj = njbuf[sl]
            nodes = node_r[pl.ds(b * At, At)]
            ni = jnp.broadcast_to(nodes[:, None, :], (At, Nbr, Fn))
            ni = ni.reshape(R, Fn)
            mb = jnp.broadcast_to(
                mask_r[pl.ds(b * At, At)][:, :, None], (At, Nbr, Fn)
            ).reshape(R, Fn)
            c2 = dot(ni * nj * mb, w2t) + b2v
            c2_scr[pl.ds(b * R, R)] = c2
            S12 = S12 + jnp.sum(c2, 0, keepdims=True)
            S22 = S22 + jnp.sum(c2 * c2, 0, keepdims=True)
            s = compute_s(b, nj)
            s4 = jnp.concatenate([s, s, s, s], axis=1)
            pnkt = dot(njtbuf[sl].astype(_BF16), w3kt)
            for mq in range(NQ):
                x = grid_x4(sl, mq, s4, pnkt)
                xk = x * keep4(mq)
                S13c = S13c + jnp.sum(xk, 0, keepdims=True)
                S23c = S23c + jnp.sum(x * xk, 0, keepdims=True)

        S13 = (S13c[:, :G] + S13c[:, G:2 * G] + S13c[:, 2 * G:3 * G]
               + S13c[:, 3 * G:])
        S23 = (S23c[:, :G] + S23c[:, G:2 * G] + S23c[:, 2 * G:3 * G]
               + S23c[:, 3 * G:])

        m2 = S12 / n2
        v2 = S22 / n2 - m2 * m2
        sc2 = g2_r[...] * lax.rsqrt(v2 + eps)
        sh2 = be2_r[...] - m2 * sc2
        m3 = S13 / n3
        v3 = S23 / n3 - m3 * m3
        sc3 = g3_r[...] * lax.rsqrt(v3 + eps)
        sh3 = be3_r[...] - m3 * sc3
        sc3t = jnp.concatenate([sc3] * 4, axis=1)
        sh3t = jnp.concatenate([sh3] * 4, axis=1)

        zs = jnp.zeros((1, Fe), _F32)
        S1s, S2s = zs, zs
        start_batch(0)
        for b in range(Bd):
            if b + 1 < Bd:
                start_batch(b + 1)
            wait_batch(b)
            sl = b % 2
            s = compute_s(b, njbuf[sl])
            s4 = jnp.concatenate([s, s, s, s], axis=1)
            pnkt = dot(njtbuf[sl].astype(_BF16), w3kt)
            tb = jnp.zeros((R, Fe), _F32)
            for mq in range(NQ):
                xn = grid_x4(sl, mq, s4, pnkt) * sc3t + sh3t
                for q in range(4):
                    gate = xn[:, q * G:q * G + Fe]
                    ext = xn[:, q * G + Fe:(q + 1) * G]
                    kq = (4 * mq + q != rowj).astype(_F32)
                    tb = tb + jax.nn.sigmoid(gate) * jnp.tanh(ext) * kq
            tbr_scr[pl.ds(b * R, R)] = tb
            S1s = S1s + jnp.sum(tb, 0, keepdims=True)
            S2s = S2s + jnp.sum(tb * tb, 0, keepdims=True)

        ms = S1s / n2
        vs = S2s / n2 - ms * ms
        scs = gs_r[...] * lax.rsqrt(vs + eps)
        shs = bes_r[...] - ms * scs

        def phase_c(b, carry):
            rows = pl.ds(b * R, R)
            c2n = c2_scr[rows] * sc2 + sh2
            twob = jax.nn.sigmoid(c2n[:, :Fe]) * jnp.tanh(c2n[:, Fe:])
            tb = tbr_scr[rows] * scs + shs
            out_r[rows] = jnp.tanh(edge_r[rows] + twob + tb)
            return carry

        lax.fori_loop(0, Bd, phase_c, 0)

    aspec = pl.BlockSpec(memory_space=pl.ANY)
    return pl.pallas_call(
        body,
        out_shape=jax.ShapeDtypeStruct((RT, Fe), _F32),
        in_specs=[pl.BlockSpec()] * 17 + [aspec, aspec],
        out_specs=pl.BlockSpec(),
        scratch_shapes=[
            pltpu.VMEM((RT, Fe), _F32),
            pltpu.VMEM((RT, G), _F32),
            pltpu.VMEM((2, NQ * R, 4 * Fe), _F32),
            pltpu.VMEM((2, R, Fn), _F32),
            pltpu.VMEM((2, R, Fn), _F32),
            pltpu.SemaphoreType.DMA((6,)),
        ],
        compiler_params=pltpu.CompilerParams(
            vmem_limit_bytes=100 * 1024 * 1024),
    )(node2, edge2, mask2, W2T, b2, W3iT, W3jT, W3kT, W3eT, W3q4,
      b3, g2, be2, g3, be3, gs, bes, njall, eg4)


def kernel(node_embedding, edge_embedding, nbr_idx, nbr_mask, W2, b2, W3,
           b3, g2, be2, g3, be3, gs, bes):
    Bd, At, Nbr, Fe = edge_embedding.shape
    Fn = node_embedding.shape[-1]
    G = 2 * Fe
    R = At * Nbr
    RT = Bd * R
    NQ = Nbr // 4

    node2 = node_embedding.reshape(Bd * At, Fn)
    edge2 = edge_embedding.reshape(RT, Fe)
    edge_perm4 = jnp.transpose(
        edge_embedding.reshape(Bd, At, NQ, 4 * Fe), (2, 0, 1, 3)
    ).reshape(NQ * Bd * At, 4 * Fe)
    idx = nbr_idx.reshape(Bd, At, Nbr).astype(jnp.int32)
    off = (jnp.arange(Bd, dtype=jnp.int32) * At)[:, None, None]
    gidx = (idx + off).reshape(Bd, R)
    gidx_t = (jnp.transpose(idx, (0, 2, 1)) + off).reshape(Bd, R)
    nidx = jnp.concatenate([gidx.reshape(RT), gidx_t.reshape(RT)])
    nidx2 = nidx.reshape(2 * RT // 100, 100)
    eidx = (jnp.arange(NQ, dtype=jnp.int32)[None, :, None] * (Bd * At)
            + gidx[:, None, :]).reshape(RT * NQ // 100, 100)

    njall, eg4 = _sc_gather(node2, edge_perm4, nidx2, eidx, 2 * RT, RT * NQ)

    mask2 = nbr_mask.reshape(Bd * At, Nbr).astype(_F32)
    W3iT = W3[:, :Fn].T
    W3jT = W3[:, Fn:2 * Fn].T.astype(_BF16)
    W3kT = W3[:, 2 * Fn:3 * Fn].T.astype(_BF16)
    W3eT = W3[:, 3 * Fn:3 * Fn + Fe].T
    W3qT = W3[:, 3 * Fn + Fe:].T
    zb = jnp.zeros((Fe, G), _F32)
    W3q4 = jnp.block([[W3qT if i == j else zb for j in range(4)]
                      for i in range(4)]).astype(_BF16)

    out2 = _tc_dense(node2, edge2, njall, eg4, mask2, W2.T,
                     b2.reshape(1, G), W3iT, W3jT, W3kT, W3eT, W3q4,
                     b3.reshape(1, G), g2.reshape(1, G), be2.reshape(1, G),
                     g3.reshape(1, G), be3.reshape(1, G),
                     gs.reshape(1, Fe), bes.reshape(1, Fe), Bd, At, Nbr)
    return out2.reshape(Bd, At, Nbr, Fe)


# speedup vs baseline: 1.3303x; 1.0095x over previous
import functools

import jax
import jax.numpy as jnp
from jax import lax
from jax.experimental import pallas as pl
from jax.experimental.pallas import tpu as pltpu
from jax.experimental.pallas import tpu_sc as plsc

_F32 = jnp.float32


def _sc_gather(node2, edge_perm4, nidx2, eidx2, nb_rows, eg_rows):
    fn = node2.shape[1]
    f4 = edge_perm4.shape[1]
    mesh = plsc.VectorSubcoreMesh(core_axis_name="c", subcore_axis_name="s")
    nw = 32
    n_chunks_n = nb_rows // (nw * 400)
    n_chunks_e = eg_rows // (nw * 400)

    @functools.partial(
        pl.kernel,
        mesh=mesh,
        out_type=[
            jax.ShapeDtypeStruct((nb_rows, fn), _F32),
            jax.ShapeDtypeStruct((eg_rows, f4), _F32),
        ],
        scratch_types=[
            pltpu.VMEM((4, 100), jnp.int32),
            pltpu.VMEM((400, fn), _F32),
            pltpu.VMEM((4, 100), jnp.int32),
            pltpu.VMEM((400, f4), _F32),
            pltpu.SemaphoreType.DMA,
        ],
        compiler_params=pltpu.CompilerParams(use_tc_tiling_on_sc=False),
    )
    def k(node_hbm, edge_hbm, nidx_hbm, eidx_hbm, nj_out, eg_out,
          idxn_v, nbuf, idxe_v, ebuf, sem):
        wid = lax.axis_index("s") * 2 + lax.axis_index("c")

        def gather_chunks(table, idx_hbm, out, idx_v, buf, n_chunks):
            def body(c, carry):
                irow = wid * (4 * n_chunks) + c * 4
                pltpu.sync_copy(idx_hbm.at[pl.ds(irow, 4)], idx_v)
                cps = [
                    pltpu.async_copy(table.at[idx_v.at[j]],
                                     buf.at[pl.ds(j * 100, 100)], sem)
                    for j in range(4)
                ]
                for cp in cps:
                    cp.wait()
                pltpu.sync_copy(buf, out.at[pl.ds(irow * 100, 400)])
                return carry
            lax.fori_loop(0, n_chunks, body, 0)

        gather_chunks(node_hbm, nidx_hbm, nj_out, idxn_v, nbuf, n_chunks_n)
        gather_chunks(edge_hbm, eidx_hbm, eg_out, idxe_v, ebuf, n_chunks_e)

    return k(node2, edge_perm4, nidx2, eidx2)


def _tc_dense(node2, edge2, njall, eg4, mask2, W2T, b2, W3iT, W3jT, W3kT,
              W3eT, W3q4, b3, g2, be2, g3, be3, gs, bes, Bd, At, Nbr):
    Fn = node2.shape[1]
    Fe = edge2.shape[1]
    G = 2 * Fe
    G4 = 4 * G
    R = At * Nbr
    RT = Bd * R
    NQ = Nbr // 4
    CH = NQ * R
    n2 = float(RT)
    n3 = float(RT * (Nbr - 1))
    eps = 1e-5

    def dot(a, b):
        return lax.dot_general(a, b, (((1,), (0,)), ((), ())),
                               preferred_element_type=_F32)

    def body(node_r, edge_r, mask_r, w2t_r, b2_r, w3it_r, w3jt_r,
             w3kt_r, w3et_r, w3q4_r, b3_r, g2_r, be2_r, g3_r, be3_r,
             gs_r, bes_r, njall_r, eg_r, out_r, tbr_scr, c2_scr, ebuf,
             njbuf, njtbuf, sems):
        w2t = w2t_r[...]
        w3it = w3it_r[...]
        w3jt = w3jt_r[...]
        w3kt = w3kt_r[...]
        w3et = w3et_r[...]
        w3q4 = w3q4_r[...]
        b2v = b2_r[...]
        b3v = b3_r[...]

        rowj = lax.broadcasted_iota(jnp.int32, (R, 1), 0) % Nbr
        laneq = lax.broadcasted_iota(jnp.int32, (R, G4), 1) // G

        def batch_copies(b):
            sl = b % 2
            return [
                pltpu.make_async_copy(njall_r.at[pl.ds(b * R, R)],
                                      njbuf.at[sl], sems.at[sl]),
                pltpu.make_async_copy(njall_r.at[pl.ds(RT + b * R, R)],
                                      njtbuf.at[sl], sems.at[2 + sl]),
                pltpu.make_async_copy(eg_r.at[pl.ds(b * CH, CH)],
                                      ebuf.at[sl], sems.at[4 + sl]),
            ]

        def start_batch(b):
            for cp in batch_copies(b):
                cp.start()

        def wait_batch(b):
            for cp in batch_copies(b):
                cp.wait()

        def compute_s(b, nj):
            nodes = node_r[pl.ds(b * At, At)]
            pi = dot(nodes, w3it)
            pib = jnp.broadcast_to(pi[:, None, :], (At, Nbr, G)).reshape(R, G)
            pe = dot(edge_r[pl.ds(b * R, R)], w3et)
            return pib + dot(nj, w3jt) + pe + b3v

        def grid_x4(sl, mq, s4, pnkt):
            egq = ebuf[sl, pl.ds(mq * R, R), :]
            epg4 = dot(egq, w3q4)
            ps = []
            for q in range(4):
                p = pnkt[(4 * mq + q) * At:(4 * mq + q + 1) * At]
                ps.append(jnp.broadcast_to(
                    p[:, None, :], (At, Nbr, G)).reshape(R, G))
            return epg4 + s4 + jnp.concatenate(ps, axis=1)

        def keep4(mq):
            return (laneq + 4 * mq != rowj).astype(_F32)

        z1 = jnp.zeros((1, G), _F32)
        z4 = jnp.zeros((1, G4), _F32)
        S12, S22, S13c, S23c = z1, z1, z4, z4
        start_batch(0)
        for b in range(Bd):
            if b + 1 < Bd:
                start_batch(b + 1)
            wait_batch(b)
            sl = b % 2
            nj = njbuf[sl]
            nodes = node_r[pl.ds(b * At, At)]
            ni = jnp.broadcast_to(nodes[:, None, :], (At, Nbr, Fn))
            ni = ni.reshape(R, Fn)
            mb = jnp.broadcast_to(
                mask_r[pl.ds(b * At, At)][:, :, None], (At, Nbr, Fn)
            ).reshape(R, Fn)
            c2 = dot(ni * nj * mb, w2t) + b2v
            c2_scr[pl.ds(b * R, R)] = c2
            S12 = S12 + jnp.sum(c2, 0, keepdims=True)
            S22 = S22 + jnp.sum(c2 * c2, 0, keepdims=True)
            s = compute_s(b, nj)
            s4 = jnp.concatenate([s, s, s, s], axis=1)
            pnkt = dot(njtbuf[sl], w3kt)
            for mq in range(NQ):
                x = grid_x4(sl, mq, s4, pnkt)
                xk = x * keep4(mq)
                S13c = S13c + jnp.sum(xk, 0, keepdims=True)
                S23c = S23c + jnp.sum(x * xk, 0, keepdims=True)

        S13 = (S13c[:, :G] + S13c[:, G:2 * G] + S13c[:, 2 * G:3 * G]
               + S13c[:, 3 * G:])
        S23 = (S23c[:, :G] + S23c[:, G:2 * G] + S23c[:, 2 * G:3 * G]
               + S23c[:, 3 * G:])

        m2 = S12 / n2
        v2 = S22 / n2 - m2 * m2
        sc2 = g2_r[...] * lax.rsqrt(v2 + eps)
        sh2 = be2_r[...] - m2 * sc2
        m3 = S13 / n3
        v3 = S23 / n3 - m3 * m3
        sc3 = g3_r[...] * lax.rsqrt(v3 + eps)
        sh3 = be3_r[...] - m3 * sc3
        sc3t = jnp.concatenate([sc3] * 4, axis=1)
        sh3t = jnp.concatenate([sh3] * 4, axis=1)

        zs = jnp.zeros((1, Fe), _F32)
        S1s, S2s = zs, zs
        start_batch(0)
        for b in range(Bd):
            if b + 1 < Bd:
                start_batch(b + 1)
            wait_batch(b)
            sl = b % 2
            s = compute_s(b, njbuf[sl])
            s4 = jnp.concatenate([s, s, s, s], axis=1)
            pnkt = dot(njtbuf[sl], w3kt)
            tb = jnp.zeros((R, Fe), _F32)
            for mq in range(NQ):
                xn = grid_x4(sl, mq, s4, pnkt) * sc3t + sh3t
                for q in range(4):
                    gate = xn[:, q * G:q * G + Fe]
                    ext = xn[:, q * G + Fe:(q + 1) * G]
                    kq = (4 * mq + q != rowj).astype(_F32)
                    tb = tb + jax.nn.sigmoid(gate) * jnp.tanh(ext) * kq
            tbr_scr[pl.ds(b * R, R)] = tb
            S1s = S1s + jnp.sum(tb, 0, keepdims=True)
            S2s = S2s + jnp.sum(tb * tb, 0, keepdims=True)

        ms = S1s / n2
        vs = S2s / n2 - ms * ms
        scs = gs_r[...] * lax.rsqrt(vs + eps)
        shs = bes_r[...] - ms * scs

        def phase_c(b, carry):
            rows = pl.ds(b * R, R)
            c2n = c2_scr[rows] * sc2 + sh2
            twob = jax.nn.sigmoid(c2n[:, :Fe]) * jnp.tanh(c2n[:, Fe:])
            tb = tbr_scr[rows] * scs + shs
            out_r[rows] = jnp.tanh(edge_r[rows] + twob + tb)
            return carry

        lax.fori_loop(0, Bd, phase_c, 0)

    aspec = pl.BlockSpec(memory_space=pl.ANY)
    return pl.pallas_call(
        body,
        out_shape=jax.ShapeDtypeStruct((RT, Fe), _F32),
        in_specs=[pl.BlockSpec()] * 17 + [aspec, aspec],
        out_specs=pl.BlockSpec(),
        scratch_shapes=[
            pltpu.VMEM((RT, Fe), _F32),
            pltpu.VMEM((RT, G), _F32),
            pltpu.VMEM((2, NQ * R, 4 * Fe), _F32),
            pltpu.VMEM((2, R, Fn), _F32),
            pltpu.VMEM((2, R, Fn), _F32),
            pltpu.SemaphoreType.DMA((6,)),
        ],
        compiler_params=pltpu.CompilerParams(
            vmem_limit_bytes=100 * 1024 * 1024),
    )(node2, edge2, mask2, W2T, b2, W3iT, W3jT, W3kT, W3eT, W3q4,
      b3, g2, be2, g3, be3, gs, bes, njall, eg4)


def kernel(node_embedding, edge_embedding, nbr_idx, nbr_mask, W2, b2, W3,
           b3, g2, be2, g3, be3, gs, bes):
    Bd, At, Nbr, Fe = edge_embedding.shape
    Fn = node_embedding.shape[-1]
    G = 2 * Fe
    R = At * Nbr
    RT = Bd * R
    NQ = Nbr // 4

    node2 = node_embedding.reshape(Bd * At, Fn)
    edge2 = edge_embedding.reshape(RT, Fe)
    edge_perm4 = jnp.transpose(
        edge_embedding.reshape(Bd, At, NQ, 4 * Fe), (2, 0, 1, 3)
    ).reshape(NQ * Bd * At, 4 * Fe)
    idx = nbr_idx.reshape(Bd, At, Nbr).astype(jnp.int32)
    off = (jnp.arange(Bd, dtype=jnp.int32) * At)[:, None, None]
    gidx = (idx + off).reshape(Bd, R)
    gidx_t = (jnp.transpose(idx, (0, 2, 1)) + off).reshape(Bd, R)
    nidx = jnp.concatenate([gidx.reshape(RT), gidx_t.reshape(RT)])
    nidx2 = nidx.reshape(2 * RT // 100, 100)
    eidx = (jnp.arange(NQ, dtype=jnp.int32)[None, :, None] * (Bd * At)
            + gidx[:, None, :]).reshape(RT * NQ // 100, 100)

    njall, eg4 = _sc_gather(node2, edge_perm4, nidx2, eidx, 2 * RT, RT * NQ)

    mask2 = nbr_mask.reshape(Bd * At, Nbr).astype(_F32)
    W3iT = W3[:, :Fn].T
    W3jT = W3[:, Fn:2 * Fn].T
    W3kT = W3[:, 2 * Fn:3 * Fn].T
    W3eT = W3[:, 3 * Fn:3 * Fn + Fe].T
    W3qT = W3[:, 3 * Fn + Fe:].T
    zb = jnp.zeros((Fe, G), _F32)
    W3q4 = jnp.block([[W3qT if i == j else zb for j in range(4)]
                      for i in range(4)])

    out2 = _tc_dense(node2, edge2, njall, eg4, mask2, W2.T,
                     b2.reshape(1, G), W3iT, W3jT, W3kT, W3eT, W3q4,
                     b3.reshape(1, G), g2.reshape(1, G), be2.reshape(1, G),
                     g3.reshape(1, G), be3.reshape(1, G),
                     gs.reshape(1, Fe), bes.reshape(1, Fe), Bd, At, Nbr)
    return out2.reshape(Bd, At, Nbr, Fe)
